# Initial kernel scaffold; baseline (speedup 1.0000x reference)
import jax
import jax.numpy as jnp
from jax.experimental import pallas as pl


def kernel(time):
    raise NotImplementedError("write your pallas kernel here")


# speedup vs baseline: 1.0454x; 1.0454x over previous
import math

import jax
import jax.numpy as jnp
from jax.experimental import pallas as pl
from jax.experimental.pallas import tpu as pltpu

_DIM = 1280
_BLK = 2048


def _emb_kernel(t_ref, o_ref):
    t = t_ref[:, :]
    i = jax.lax.broadcasted_iota(jnp.int32, (1, _DIM), 1)
    power = (2.0 / _DIM) * (i // 2).astype(jnp.float32)
    rate = jnp.exp(power * (-math.log(10000.0)))
    phase = (i % 2).astype(jnp.float32) * (math.pi / 2.0)
    o_ref[:, :] = jnp.sin(t * rate + phase)


def kernel(time):
    b = time.shape[0]
    t2 = time.reshape(b, 1)
    return pl.pallas_call(
        _emb_kernel,
        grid=(b // _BLK,),
        in_specs=[pl.BlockSpec((_BLK, 1), lambda g: (g, 0))],
        out_specs=pl.BlockSpec((_BLK, _DIM), lambda g: (g, 0)),
        out_shape=jax.ShapeDtypeStruct((b, _DIM), jnp.float32),
        compiler_params=pltpu.CompilerParams(
            dimension_semantics=("parallel",),
        ),
    )(t2)


# speedup vs baseline: 1.2322x; 1.1787x over previous
import math

import jax
import jax.numpy as jnp
from jax.experimental import pallas as pl
from jax.experimental.pallas import tpu as pltpu

_DIM = 1280
_BLK = 512


_PI2_H = 1.5703125
_PI2_M = 0.0004837512969970703
_PI2_L = 7.549790126404332e-08
_TWO_OVER_PI = 2.0 / math.pi


_ROWS = 8


def _emb_kernel(t_ref, o_ref):
    i = jax.lax.broadcasted_iota(jnp.int32, (1, _DIM), 1)
    power = (2.0 / _DIM) * (i // 2).astype(jnp.float32)
    rate = jnp.exp(power * (-math.log(10000.0)))
    parity = i & 1

    def body(j, carry):
        t = t_ref[pl.ds(j * _ROWS, _ROWS), :]
        ang = t * rate
        kf = jnp.round(ang * _TWO_OVER_PI)
        r = ang - kf * _PI2_H
        r = r - kf * _PI2_M
        r = r - kf * _PI2_L
        k = kf.astype(jnp.int32) + parity
        r2 = r * r
        s = r + r * r2 * (-1.0 / 6.0 + r2 * (1.0 / 120.0 + r2 * (-1.0 / 5040.0)))
        c = 1.0 + r2 * (-0.5 + r2 * (1.0 / 24.0 + r2 * (-1.0 / 720.0)))
        v = jnp.where((k & 1) == 1, c, s)
        o_ref[pl.ds(j * _ROWS, _ROWS), :] = jnp.where((k & 2) == 2, -v, v)
        return carry

    jax.lax.fori_loop(0, _BLK // _ROWS, body, 0)


def kernel(time):
    b = time.shape[0]
    t2 = time.reshape(b, 1)
    return pl.pallas_call(
        _emb_kernel,
        grid=(b // _BLK,),
        in_specs=[pl.BlockSpec((_BLK, 1), lambda g: (g, 0))],
        out_specs=pl.BlockSpec((_BLK, _DIM), lambda g: (g, 0)),
        out_shape=jax.ShapeDtypeStruct((b, _DIM), jnp.float32),
        compiler_params=pltpu.CompilerParams(
            dimension_semantics=("parallel",),
        ),
    )(t2)


# speedup vs baseline: 3.1609x; 2.5653x over previous
import math

import jax
import jax.numpy as jnp
from jax.experimental import pallas as pl
from jax.experimental.pallas import tpu as pltpu

_DIM = 1280
_BLK = 512


_PI2_H = 1.5703125
_PI2_M = 0.0004837512969970703
_PI2_L = 7.549790126404332e-08
_TWO_OVER_PI = 2.0 / math.pi


_ROWS = 8


def _emb_kernel(t_ref, o_ref):
    i = jax.lax.broadcasted_iota(jnp.int32, (1, _DIM), 1)
    power = (2.0 / _DIM) * (i // 2).astype(jnp.float32)
    rate = jnp.exp(power * (-math.log(10000.0)))
    parity = i & 1

    def body(j):
        t = t_ref[pl.ds(j * _ROWS, _ROWS), :]
        ang = t * rate
        kf = jnp.round(ang * _TWO_OVER_PI)
        r = ang - kf * _PI2_H
        r = r - kf * _PI2_M
        r = r - kf * _PI2_L
        k = kf.astype(jnp.int32) + parity
        r2 = r * r
        s = r + r * r2 * (-1.0 / 6.0 + r2 * (1.0 / 120.0 + r2 * (-1.0 / 5040.0)))
        c = 1.0 + r2 * (-0.5 + r2 * (1.0 / 24.0 + r2 * (-1.0 / 720.0)))
        v = jnp.where((k & 1) == 1, c, s)
        o_ref[pl.ds(j * _ROWS, _ROWS), :] = jnp.where((k & 2) == 2, -v, v)

    for j in range(_BLK // _ROWS):
        body(j)


def kernel(time):
    b = time.shape[0]
    t2 = time.reshape(b, 1)
    return pl.pallas_call(
        _emb_kernel,
        grid=(b // _BLK,),
        in_specs=[pl.BlockSpec((_BLK, 1), lambda g: (g, 0))],
        out_specs=pl.BlockSpec((_BLK, _DIM), lambda g: (g, 0)),
        out_shape=jax.ShapeDtypeStruct((b, _DIM), jnp.float32),
        compiler_params=pltpu.CompilerParams(
            dimension_semantics=("parallel",),
        ),
    )(t2)


# speedup vs baseline: 4.2177x; 1.3344x over previous
import math

import jax
import jax.numpy as jnp
from jax.experimental import pallas as pl
from jax.experimental.pallas import tpu as pltpu

_DIM = 1280
_BLK = 512


_TWO_OVER_PI = 2.0 / math.pi
_PI_OVER_TWO = math.pi / 2.0


_ROWS = 8


def _emb_kernel(t_ref, o_ref):
    i = jax.lax.broadcasted_iota(jnp.int32, (1, _DIM), 1)
    power = (2.0 / _DIM) * (i // 2).astype(jnp.float32)
    rate = jnp.exp(power * (-math.log(10000.0)))
    rate_q = rate * _TWO_OVER_PI
    parity = i & 1

    def body(j):
        t = t_ref[pl.ds(j * _ROWS, _ROWS), :]
        u = t * rate_q
        kf = jnp.round(u)
        r = (u - kf) * _PI_OVER_TWO
        k = kf.astype(jnp.int32) + parity
        r2 = r * r
        s = r + r * r2 * (-1.0 / 6.0 + r2 * (1.0 / 120.0))
        c = 1.0 + r2 * (-0.5 + r2 * (1.0 / 24.0))
        v = jnp.where((k & 1) == 1, c, s)
        o_ref[pl.ds(j * _ROWS, _ROWS), :] = jnp.where((k & 2) == 2, -v, v)

    for j in range(_BLK // _ROWS):
        body(j)


def kernel(time):
    b = time.shape[0]
    t2 = time.reshape(b, 1)
    return pl.pallas_call(
        _emb_kernel,
        grid=(b // _BLK,),
        in_specs=[pl.BlockSpec((_BLK, 1), lambda g: (g, 0))],
        out_specs=pl.BlockSpec((_BLK, _DIM), lambda g: (g, 0)),
        out_shape=jax.ShapeDtypeStruct((b, _DIM), jnp.float32),
        compiler_params=pltpu.CompilerParams(
            dimension_semantics=("parallel",),
        ),
    )(t2)
